# Initial kernel scaffold; baseline (speedup 1.0000x reference)
import jax
import jax.numpy as jnp
from jax.experimental import pallas as pl


def kernel(x, e, W1, W2, W3, W4, W5, bn_x_gamma, bn_x_beta, bn_e_gamma, bn_e_beta, edge_index):
    raise NotImplementedError("write your pallas kernel here")


# speedup vs baseline: 1.3603x; 1.3603x over previous
import functools

import jax
import jax.numpy as jnp
from jax import lax
from jax.experimental import pallas as pl
from jax.experimental.pallas import tpu as pltpu
from jax.experimental.pallas import tpu_sc as plsc

N_NODES = 10000
N_EDGES = 320000
H = 128
EPS = 1e-5

NC = 2
NS = 16
NW = NC * NS
EPW = N_EDGES // NW
CHUNK = 80
N_CHUNKS = EPW // CHUNK
ZROWS = 25

NODE_BLK = 1000
EDGE_BLK = 4000


def _tables_body(x_ref, w25_ref, w4t_ref, w1t_ref, t25_ref, t4_ref, t1_ref):
    xb = x_ref[...]
    t25_ref[...] = jnp.dot(xb, w25_ref[...], preferred_element_type=jnp.float32)
    t4_ref[...] = jnp.dot(xb, w4t_ref[...], preferred_element_type=jnp.float32)
    t1_ref[...] = jnp.dot(xb, w1t_ref[...], preferred_element_type=jnp.float32)


def _make_tables(x, w25, w4t, w1t):
    grid = N_NODES // NODE_BLK
    return pl.pallas_call(
        _tables_body,
        grid=(grid,),
        in_specs=[
            pl.BlockSpec((NODE_BLK, H), lambda i: (i, 0)),
            pl.BlockSpec((H, 2 * H), lambda i: (0, 0)),
            pl.BlockSpec((H, H), lambda i: (0, 0)),
            pl.BlockSpec((H, H), lambda i: (0, 0)),
        ],
        out_specs=[
            pl.BlockSpec((NODE_BLK, 2 * H), lambda i: (i, 0)),
            pl.BlockSpec((NODE_BLK, H), lambda i: (i, 0)),
            pl.BlockSpec((NODE_BLK, H), lambda i: (i, 0)),
        ],
        out_shape=[
            jax.ShapeDtypeStruct((N_NODES, 2 * H), jnp.float32),
            jax.ShapeDtypeStruct((N_NODES, H), jnp.float32),
            jax.ShapeDtypeStruct((N_NODES, H), jnp.float32),
        ],
    )(x, w25, w4t, w1t)


def _sc_edge_body(t25_hbm, t4_hbm, e_hbm, src_hbm, dst_hbm,
                  g45_hbm, aggr_hbm,
                  idx_s, idx_d, e_buf, t25_buf, t4_buf, zbuf, aggr_sh,
                  sem_a, sem_b, sem_c):
    cid = lax.axis_index("c")
    sid = lax.axis_index("s")
    wid = sid * NC + cid

    def _zrow(r, carry):
        for j in range(8):
            zbuf[r, pl.ds(j * 16, 16)] = jnp.zeros((16,), jnp.float32)
        return carry
    lax.fori_loop(0, ZROWS, _zrow, 0)
    for i in range(625 // ZROWS):
        pltpu.sync_copy(zbuf, aggr_sh.at[pl.ds(sid * 625 + i * ZROWS, ZROWS)])
    plsc.subcore_barrier()

    base0 = wid * EPW

    def _chunk(j, carry):
        base = base0 + j * CHUNK
        pltpu.sync_copy(src_hbm.at[pl.ds(base, CHUNK)], idx_s)
        pltpu.sync_copy(dst_hbm.at[pl.ds(base, CHUNK)], idx_d)
        cp1 = pltpu.async_copy(t25_hbm.at[idx_s], t25_buf, sem_a)
        cp2 = pltpu.async_copy(t4_hbm.at[idx_d], t4_buf, sem_b)
        cp3 = pltpu.async_copy(e_hbm.at[pl.ds(base, CHUNK)], e_buf, sem_c)
        cp1.wait()
        cp2.wait()
        cp3.wait()

        def _row(r, rc):
            for jj in range(8):
                sl = pl.ds(jj * 16, 16)
                e_buf[r, sl] = e_buf[r, sl] * t25_buf[r, sl]
                t4_buf[r, sl] = t4_buf[r, sl] + t25_buf[r, pl.ds(H + jj * 16, 16)]
            return rc
        lax.fori_loop(0, CHUNK, _row, 0)

        pltpu.sync_copy(t4_buf, g45_hbm.at[pl.ds(base, CHUNK)])
        pltpu.sync_copy(e_buf, aggr_sh.at[idx_d], add=True)
        return carry
    lax.fori_loop(0, N_CHUNKS, _chunk, 0)

    plsc.subcore_barrier()

    @pl.when(sid == 0)
    def _():
        pltpu.sync_copy(aggr_sh, aggr_hbm.at[cid])


def _run_sc_edge(t25, t4, e, src, dst):
    mesh = plsc.VectorSubcoreMesh(core_axis_name="c", subcore_axis_name="s",
                                  num_cores=NC, num_subcores=NS)
    fn = pl.kernel(
        _sc_edge_body,
        out_type=[
            jax.ShapeDtypeStruct((N_EDGES, H), jnp.float32),
            jax.ShapeDtypeStruct((NC, N_NODES, H), jnp.float32),
        ],
        mesh=mesh,
        scratch_types=[
            pltpu.VMEM((CHUNK,), jnp.int32),
            pltpu.VMEM((CHUNK,), jnp.int32),
            pltpu.VMEM((CHUNK, H), jnp.float32),
            pltpu.VMEM((CHUNK, 2 * H), jnp.float32),
            pltpu.VMEM((CHUNK, H), jnp.float32),
            pltpu.VMEM((ZROWS, H), jnp.float32),
            pltpu.VMEM_SHARED((N_NODES, H), jnp.float32),
            pltpu.SemaphoreType.DMA,
            pltpu.SemaphoreType.DMA,
            pltpu.SemaphoreType.DMA,
        ],
    )
    return fn(t25, t4, e, src, dst)


def _e_stats_body(e_ref, g45_ref, w3t_ref, sums_ref):
    i = pl.program_id(0)
    h = jnp.dot(e_ref[...], w3t_ref[...],
                preferred_element_type=jnp.float32) + g45_ref[...]

    @pl.when(i == 0)
    def _():
        sums_ref[...] = jnp.zeros_like(sums_ref)

    sums_ref[0, :] += jnp.sum(h, axis=0)
    sums_ref[1, :] += jnp.sum(h * h, axis=0)


def _e_stats(e, g45, w3t):
    grid = N_EDGES // EDGE_BLK
    return pl.pallas_call(
        _e_stats_body,
        grid=(grid,),
        in_specs=[
            pl.BlockSpec((EDGE_BLK, H), lambda i: (i, 0)),
            pl.BlockSpec((EDGE_BLK, H), lambda i: (i, 0)),
            pl.BlockSpec((H, H), lambda i: (0, 0)),
        ],
        out_specs=pl.BlockSpec((8, H), lambda i: (0, 0)),
        out_shape=jax.ShapeDtypeStruct((8, H), jnp.float32),
    )(e, g45, w3t)


def _e_apply_body(e_ref, g45_ref, w3t_ref, sums_ref, gamma_ref, beta_ref,
                  out_ref):
    h = jnp.dot(e_ref[...], w3t_ref[...],
                preferred_element_type=jnp.float32) + g45_ref[...]
    inv_n = 1.0 / N_EDGES
    mean = sums_ref[0:1, :] * inv_n
    var = sums_ref[1:2, :] * inv_n - mean * mean
    scale = gamma_ref[...] * lax.rsqrt(var + EPS)
    shift = beta_ref[...] - mean * scale
    out_ref[...] = e_ref[...] + jnp.maximum(h * scale + shift, 0.0)


def _e_apply(e, g45, w3t, sums, gamma, beta):
    grid = N_EDGES // EDGE_BLK
    return pl.pallas_call(
        _e_apply_body,
        grid=(grid,),
        in_specs=[
            pl.BlockSpec((EDGE_BLK, H), lambda i: (i, 0)),
            pl.BlockSpec((EDGE_BLK, H), lambda i: (i, 0)),
            pl.BlockSpec((H, H), lambda i: (0, 0)),
            pl.BlockSpec((8, H), lambda i: (0, 0)),
            pl.BlockSpec((1, H), lambda i: (0, 0)),
            pl.BlockSpec((1, H), lambda i: (0, 0)),
        ],
        out_specs=pl.BlockSpec((EDGE_BLK, H), lambda i: (i, 0)),
        out_shape=jax.ShapeDtypeStruct((N_EDGES, H), jnp.float32),
    )(e, g45, w3t, sums, gamma, beta)


def _x_stats_body(t1_ref, aggr_ref, sums_ref):
    i = pl.program_id(0)
    h = t1_ref[...] + aggr_ref[0] + aggr_ref[1]

    @pl.when(i == 0)
    def _():
        sums_ref[...] = jnp.zeros_like(sums_ref)

    sums_ref[0, :] += jnp.sum(h, axis=0)
    sums_ref[1, :] += jnp.sum(h * h, axis=0)


def _x_stats(t1, aggr):
    grid = N_NODES // NODE_BLK
    return pl.pallas_call(
        _x_stats_body,
        grid=(grid,),
        in_specs=[
            pl.BlockSpec((NODE_BLK, H), lambda i: (i, 0)),
            pl.BlockSpec((NC, NODE_BLK, H), lambda i: (0, i, 0)),
        ],
        out_specs=pl.BlockSpec((8, H), lambda i: (0, 0)),
        out_shape=jax.ShapeDtypeStruct((8, H), jnp.float32),
    )(t1, aggr)


def _x_apply_body(x_ref, t1_ref, aggr_ref, sums_ref, gamma_ref, beta_ref,
                  out_ref):
    h = t1_ref[...] + aggr_ref[0] + aggr_ref[1]
    inv_n = 1.0 / N_NODES
    mean = sums_ref[0:1, :] * inv_n
    var = sums_ref[1:2, :] * inv_n - mean * mean
    scale = gamma_ref[...] * lax.rsqrt(var + EPS)
    shift = beta_ref[...] - mean * scale
    out_ref[...] = x_ref[...] + jnp.maximum(h * scale + shift, 0.0)


def _x_apply(x, t1, aggr, sums, gamma, beta):
    grid = N_NODES // NODE_BLK
    return pl.pallas_call(
        _x_apply_body,
        grid=(grid,),
        in_specs=[
            pl.BlockSpec((NODE_BLK, H), lambda i: (i, 0)),
            pl.BlockSpec((NODE_BLK, H), lambda i: (i, 0)),
            pl.BlockSpec((NC, NODE_BLK, H), lambda i: (0, i, 0)),
            pl.BlockSpec((8, H), lambda i: (0, 0)),
            pl.BlockSpec((1, H), lambda i: (0, 0)),
            pl.BlockSpec((1, H), lambda i: (0, 0)),
        ],
        out_specs=pl.BlockSpec((NODE_BLK, H), lambda i: (i, 0)),
        out_shape=jax.ShapeDtypeStruct((N_NODES, H), jnp.float32),
    )(x, t1, aggr, sums, gamma, beta)


def kernel(x, e, W1, W2, W3, W4, W5, bn_x_gamma, bn_x_beta, bn_e_gamma,
           bn_e_beta, edge_index):
    src = edge_index[0].astype(jnp.int32)
    dst = edge_index[1].astype(jnp.int32)

    w25 = jnp.concatenate([W2.T, W5.T], axis=1)
    w4t = W4.T
    w1t = W1.T
    w3t = W3.T

    t25, t4, t1 = _make_tables(x, w25, w4t, w1t)
    g45, aggr = _run_sc_edge(t25, t4, e, src, dst)

    e_sums = _e_stats(e, g45, w3t)
    e_new = _e_apply(e, g45, w3t, e_sums,
                     bn_e_gamma.reshape(1, H), bn_e_beta.reshape(1, H))

    x_sums = _x_stats(t1, aggr)
    x_new = _x_apply(x, t1, aggr, x_sums,
                     bn_x_gamma.reshape(1, H), bn_x_beta.reshape(1, H))
    return (x_new, e_new)


# speedup vs baseline: 1.4683x; 1.0794x over previous
import jax
import jax.numpy as jnp
from jax import lax
from jax.experimental import pallas as pl
from jax.experimental.pallas import tpu as pltpu
from jax.experimental.pallas import tpu_sc as plsc

N_NODES = 10000
N_EDGES = 320000
H = 128
EPS = 1e-5

NC = 2
NS = 16
EPT = N_EDGES // NS
CHUNK = 80
CPT = EPT // CHUNK
ZROWS = 25

NODE_BLK = 1000
EDGE_BLK = 4000


def _tables_body(x_ref, w2t_ref, w5t_ref, w4t_ref, w1t_ref,
                 ta_ref, t4_ref, t1_ref):
    xb = x_ref[...]
    ta_ref[0] = jnp.dot(xb, w2t_ref[...], preferred_element_type=jnp.float32)
    ta_ref[1] = jnp.dot(xb, w5t_ref[...], preferred_element_type=jnp.float32)
    t4_ref[...] = jnp.dot(xb, w4t_ref[...], preferred_element_type=jnp.float32)
    t1_ref[...] = jnp.dot(xb, w1t_ref[...], preferred_element_type=jnp.float32)


def _make_tables(x, w2t, w5t, w4t, w1t):
    grid = N_NODES // NODE_BLK
    return pl.pallas_call(
        _tables_body,
        grid=(grid,),
        in_specs=[
            pl.BlockSpec((NODE_BLK, H), lambda i: (i, 0)),
            pl.BlockSpec((H, H), lambda i: (0, 0)),
            pl.BlockSpec((H, H), lambda i: (0, 0)),
            pl.BlockSpec((H, H), lambda i: (0, 0)),
            pl.BlockSpec((H, H), lambda i: (0, 0)),
        ],
        out_specs=[
            pl.BlockSpec((NC, NODE_BLK, H), lambda i: (0, i, 0)),
            pl.BlockSpec((NODE_BLK, H), lambda i: (i, 0)),
            pl.BlockSpec((NODE_BLK, H), lambda i: (i, 0)),
        ],
        out_shape=[
            jax.ShapeDtypeStruct((NC, N_NODES, H), jnp.float32),
            jax.ShapeDtypeStruct((N_NODES, H), jnp.float32),
            jax.ShapeDtypeStruct((N_NODES, H), jnp.float32),
        ],
    )(x, w2t, w5t, w4t, w1t)


def _sc_edge_body(ta, t4tab, e2d, aidx2, bidx2, didx2, g45o, aggro,
                  ia0, ib0, id0, ab0, bb0,
                  ia1, ib1, id1, ab1, bb1,
                  zbuf, aggr_sh,
                  isem0, isem1, gsem0, gsem1, ssem0, ssem1):
    cid = lax.axis_index("c")
    sid = lax.axis_index("s")
    is_c0 = cid == 0

    sets = (
        dict(ia=ia0, ib=ib0, id=id0, a=ab0, b=bb0, im=isem0, g=gsem0, s=ssem0),
        dict(ia=ia1, ib=ib1, id=id1, a=ab1, b=bb1, im=isem1, g=gsem1, s=ssem1),
    )

    @pl.when(is_c0)
    def _():
        def _zrow(r, carry):
            for j in range(H // 16):
                zbuf[r, pl.ds(j * 16, 16)] = jnp.zeros((16,), jnp.float32)
            return carry
        lax.fori_loop(0, ZROWS, _zrow, 0)
        for i in range(625 // ZROWS):
            pltpu.sync_copy(zbuf, aggr_sh.at[pl.ds(sid * 625 + i * ZROWS, ZROWS)])
    plsc.subcore_barrier()

    ibase = cid * N_EDGES + sid * EPT
    ebase = sid * EPT

    def body(j, carry):
        idescs = []
        for k in range(2):
            S = sets[k]
            off = ibase + (2 * j + k) * CHUNK
            idescs.append((
                pltpu.async_copy(aidx2.at[pl.ds(off, CHUNK)], S["ia"], S["im"]),
                pltpu.async_copy(bidx2.at[pl.ds(off, CHUNK)], S["ib"], S["im"]),
                pltpu.async_copy(didx2.at[pl.ds(off, CHUNK)], S["id"], S["im"]),
            ))
        gd = []
        for k in range(2):
            S = sets[k]
            for d in idescs[k]:
                d.wait()
            gd.append(pltpu.async_copy(ta.at[S["ia"]], S["a"], S["g"]))

            @pl.when(is_c0)
            def _(S=S, k=k):
                pltpu.async_copy(
                    e2d.at[pl.ds(ebase + (2 * j + k) * CHUNK, CHUNK)],
                    S["b"], S["g"])

            @pl.when(~is_c0)
            def _(S=S):
                pltpu.async_copy(t4tab.at[S["ib"]], S["b"], S["g"])

        for k in range(2):
            S = sets[k]
            gd[k].wait()

            @pl.when(is_c0)
            def _(S=S):
                pltpu.make_async_copy(e2d.at[pl.ds(0, CHUNK)], S["b"],
                                      S["g"]).wait()

            @pl.when(~is_c0)
            def _(S=S):
                pltpu.make_async_copy(t4tab.at[S["ib"]], S["b"], S["g"]).wait()

            av, bv = S["a"], S["b"]

            def _row(r, carry2):
                for kk in range(H // 16):
                    sl = pl.ds(kk * 16, 16)
                    a = av[r, sl]
                    b = bv[r, sl]
                    bv[r, sl] = jnp.where(is_c0, a * b, a + b)
                return carry2
            lax.fori_loop(0, CHUNK, _row, 0, unroll=2)

            @pl.when(is_c0)
            def _(S=S):
                pltpu.async_copy(S["b"], aggr_sh.at[S["id"]], S["s"], add=True)

            @pl.when(~is_c0)
            def _(S=S, k=k):
                pltpu.async_copy(
                    S["b"], g45o.at[pl.ds(ebase + (2 * j + k) * CHUNK, CHUNK)],
                    S["s"])

        for k in range(2):
            S = sets[k]

            @pl.when(is_c0)
            def _(S=S):
                pltpu.make_async_copy(S["b"], aggr_sh.at[S["id"]],
                                      S["s"]).wait()

            @pl.when(~is_c0)
            def _(S=S):
                pltpu.make_async_copy(S["b"], g45o.at[pl.ds(0, CHUNK)],
                                      S["s"]).wait()
        return carry
    lax.fori_loop(0, CPT // 2, body, 0)

    plsc.subcore_barrier()

    @pl.when(is_c0 & (sid == 0))
    def _():
        pltpu.sync_copy(aggr_sh, aggro)


def _run_sc_edge(ta, t4tab, e2d, aidx2, bidx2, didx2):
    mesh = plsc.VectorSubcoreMesh(core_axis_name="c", subcore_axis_name="s",
                                  num_cores=NC, num_subcores=NS)
    buf_set = [
        pltpu.VMEM((CHUNK,), jnp.int32),
        pltpu.VMEM((CHUNK,), jnp.int32),
        pltpu.VMEM((CHUNK,), jnp.int32),
        pltpu.VMEM((CHUNK, H), jnp.float32),
        pltpu.VMEM((CHUNK, H), jnp.float32),
    ]
    fn = pl.kernel(
        _sc_edge_body,
        out_type=[
            jax.ShapeDtypeStruct((N_EDGES, H), jnp.float32),
            jax.ShapeDtypeStruct((N_NODES, H), jnp.float32),
        ],
        mesh=mesh,
        scratch_types=(buf_set * 2) + [
            pltpu.VMEM((ZROWS, H), jnp.float32),
            pltpu.VMEM_SHARED((N_NODES, H), jnp.float32),
            pltpu.SemaphoreType.DMA,
            pltpu.SemaphoreType.DMA,
            pltpu.SemaphoreType.DMA,
            pltpu.SemaphoreType.DMA,
            pltpu.SemaphoreType.DMA,
            pltpu.SemaphoreType.DMA,
        ],
    )
    return fn(ta, t4tab, e2d, aidx2, bidx2, didx2)


def _e_stats_body(e_ref, g45_ref, w3t_ref, sums_ref):
    i = pl.program_id(0)
    h = jnp.dot(e_ref[...], w3t_ref[...],
                preferred_element_type=jnp.float32) + g45_ref[...]

    @pl.when(i == 0)
    def _():
        sums_ref[...] = jnp.zeros_like(sums_ref)

    sums_ref[0, :] += jnp.sum(h, axis=0)
    sums_ref[1, :] += jnp.sum(h * h, axis=0)


def _e_stats(e, g45, w3t):
    grid = N_EDGES // EDGE_BLK
    return pl.pallas_call(
        _e_stats_body,
        grid=(grid,),
        in_specs=[
            pl.BlockSpec((EDGE_BLK, H), lambda i: (i, 0)),
            pl.BlockSpec((EDGE_BLK, H), lambda i: (i, 0)),
            pl.BlockSpec((H, H), lambda i: (0, 0)),
        ],
        out_specs=pl.BlockSpec((8, H), lambda i: (0, 0)),
        out_shape=jax.ShapeDtypeStruct((8, H), jnp.float32),
    )(e, g45, w3t)


def _e_apply_body(e_ref, g45_ref, w3t_ref, sums_ref, gamma_ref, beta_ref,
                  out_ref):
    h = jnp.dot(e_ref[...], w3t_ref[...],
                preferred_element_type=jnp.float32) + g45_ref[...]
    inv_n = 1.0 / N_EDGES
    mean = sums_ref[0:1, :] * inv_n
    var = sums_ref[1:2, :] * inv_n - mean * mean
    scale = gamma_ref[...] * lax.rsqrt(var + EPS)
    shift = beta_ref[...] - mean * scale
    out_ref[...] = e_ref[...] + jnp.maximum(h * scale + shift, 0.0)


def _e_apply(e, g45, w3t, sums, gamma, beta):
    grid = N_EDGES // EDGE_BLK
    return pl.pallas_call(
        _e_apply_body,
        grid=(grid,),
        in_specs=[
            pl.BlockSpec((EDGE_BLK, H), lambda i: (i, 0)),
            pl.BlockSpec((EDGE_BLK, H), lambda i: (i, 0)),
            pl.BlockSpec((H, H), lambda i: (0, 0)),
            pl.BlockSpec((8, H), lambda i: (0, 0)),
            pl.BlockSpec((1, H), lambda i: (0, 0)),
            pl.BlockSpec((1, H), lambda i: (0, 0)),
        ],
        out_specs=pl.BlockSpec((EDGE_BLK, H), lambda i: (i, 0)),
        out_shape=jax.ShapeDtypeStruct((N_EDGES, H), jnp.float32),
    )(e, g45, w3t, sums, gamma, beta)


def _x_stats_body(t1_ref, aggr_ref, sums_ref):
    i = pl.program_id(0)
    h = t1_ref[...] + aggr_ref[...]

    @pl.when(i == 0)
    def _():
        sums_ref[...] = jnp.zeros_like(sums_ref)

    sums_ref[0, :] += jnp.sum(h, axis=0)
    sums_ref[1, :] += jnp.sum(h * h, axis=0)


def _x_stats(t1, aggr):
    grid = N_NODES // NODE_BLK
    return pl.pallas_call(
        _x_stats_body,
        grid=(grid,),
        in_specs=[
            pl.BlockSpec((NODE_BLK, H), lambda i: (i, 0)),
            pl.BlockSpec((NODE_BLK, H), lambda i: (i, 0)),
        ],
        out_specs=pl.BlockSpec((8, H), lambda i: (0, 0)),
        out_shape=jax.ShapeDtypeStruct((8, H), jnp.float32),
    )(t1, aggr)


def _x_apply_body(x_ref, t1_ref, aggr_ref, sums_ref, gamma_ref, beta_ref,
                  out_ref):
    h = t1_ref[...] + aggr_ref[...]
    inv_n = 1.0 / N_NODES
    mean = sums_ref[0:1, :] * inv_n
    var = sums_ref[1:2, :] * inv_n - mean * mean
    scale = gamma_ref[...] * lax.rsqrt(var + EPS)
    shift = beta_ref[...] - mean * scale
    out_ref[...] = x_ref[...] + jnp.maximum(h * scale + shift, 0.0)


def _x_apply(x, t1, aggr, sums, gamma, beta):
    grid = N_NODES // NODE_BLK
    return pl.pallas_call(
        _x_apply_body,
        grid=(grid,),
        in_specs=[
            pl.BlockSpec((NODE_BLK, H), lambda i: (i, 0)),
            pl.BlockSpec((NODE_BLK, H), lambda i: (i, 0)),
            pl.BlockSpec((NODE_BLK, H), lambda i: (i, 0)),
            pl.BlockSpec((8, H), lambda i: (0, 0)),
            pl.BlockSpec((1, H), lambda i: (0, 0)),
            pl.BlockSpec((1, H), lambda i: (0, 0)),
        ],
        out_specs=pl.BlockSpec((NODE_BLK, H), lambda i: (i, 0)),
        out_shape=jax.ShapeDtypeStruct((N_NODES, H), jnp.float32),
    )(x, t1, aggr, sums, gamma, beta)


def kernel(x, e, W1, W2, W3, W4, W5, bn_x_gamma, bn_x_beta, bn_e_gamma,
           bn_e_beta, edge_index):
    src = edge_index[0].astype(jnp.int32)
    dst = edge_index[1].astype(jnp.int32)

    eye = lax.iota(jnp.int32, N_EDGES)
    aidx2 = jnp.concatenate([src, src + N_NODES])
    bidx2 = jnp.concatenate([eye, dst])
    didx2 = jnp.concatenate([dst, dst])

    ta3, t4, t1 = _make_tables(x, W2.T, W5.T, W4.T, W1.T)
    g45, aggr = _run_sc_edge(ta3.reshape(NC * N_NODES, H), t4, e,
                             aidx2, bidx2, didx2)

    w3t = W3.T
    e_sums = _e_stats(e, g45, w3t)
    e_new = _e_apply(e, g45, w3t, e_sums,
                     bn_e_gamma.reshape(1, H), bn_e_beta.reshape(1, H))

    x_sums = _x_stats(t1, aggr)
    x_new = _x_apply(x, t1, aggr, x_sums,
                     bn_x_gamma.reshape(1, H), bn_x_beta.reshape(1, H))
    return (x_new, e_new)


# speedup vs baseline: 1.5095x; 1.0280x over previous
import jax
import jax.numpy as jnp
from jax import lax
from jax.experimental import pallas as pl
from jax.experimental.pallas import tpu as pltpu
from jax.experimental.pallas import tpu_sc as plsc

N_NODES = 10000
N_EDGES = 320000
H = 128
EPS = 1e-5

NC = 2
NS = 16
EPT = N_EDGES // NS
CHUNK = 80
CPT = EPT // CHUNK
EPG = 4000
BPG = EPG // (2 * CHUNK)
NGRP = EPT // EPG
ZROWS = 5

NODE_BLK = 1000
EDGE_BLK = 4000


def _tables_body(x_ref, w2t_ref, w5t_ref, w4t_ref, w1t_ref,
                 ta_ref, t4_ref, t1_ref):
    xb = x_ref[...]
    ta_ref[0] = jnp.dot(xb, w2t_ref[...], preferred_element_type=jnp.float32)
    ta_ref[1] = jnp.dot(xb, w5t_ref[...], preferred_element_type=jnp.float32)
    t4_ref[...] = jnp.dot(xb, w4t_ref[...], preferred_element_type=jnp.float32)
    t1_ref[...] = jnp.dot(xb, w1t_ref[...], preferred_element_type=jnp.float32)


def _make_tables(x, w2t, w5t, w4t, w1t):
    grid = N_NODES // NODE_BLK
    return pl.pallas_call(
        _tables_body,
        grid=(grid,),
        in_specs=[
            pl.BlockSpec((NODE_BLK, H), lambda i: (i, 0)),
            pl.BlockSpec((H, H), lambda i: (0, 0)),
            pl.BlockSpec((H, H), lambda i: (0, 0)),
            pl.BlockSpec((H, H), lambda i: (0, 0)),
            pl.BlockSpec((H, H), lambda i: (0, 0)),
        ],
        out_specs=[
            pl.BlockSpec((NC, NODE_BLK, H), lambda i: (0, i, 0)),
            pl.BlockSpec((NODE_BLK, H), lambda i: (i, 0)),
            pl.BlockSpec((NODE_BLK, H), lambda i: (i, 0)),
        ],
        out_shape=[
            jax.ShapeDtypeStruct((NC, N_NODES, H), jnp.float32),
            jax.ShapeDtypeStruct((N_NODES, H), jnp.float32),
            jax.ShapeDtypeStruct((N_NODES, H), jnp.float32),
        ],
    )(x, w2t, w5t, w4t, w1t)


def _sc_edge_body(ta, t4tab, e2d, aidx2, dst1, g45o, aggro,
                  id0, ab0, bb0,
                  id1, ab1, bb1,
                  big_a, big_d, zbuf, aggr_sh,
                  gsem0, gsem1, ssem0, ssem1, isem0, isem1):
    cid = lax.axis_index("c")
    sid = lax.axis_index("s")
    is_c0 = cid == 0

    sets = (
        dict(id=id0, a=ab0, b=bb0, g=gsem0, s=ssem0, im=isem0),
        dict(id=id1, a=ab1, b=bb1, g=gsem1, s=ssem1, im=isem1),
    )

    @pl.when(is_c0)
    def _():
        def _zrow(r, carry):
            for j in range(H // 16):
                zbuf[r, pl.ds(j * 16, 16)] = jnp.zeros((16,), jnp.float32)
            return carry
        lax.fori_loop(0, ZROWS, _zrow, 0)
        for i in range(625 // ZROWS):
            pltpu.sync_copy(zbuf, aggr_sh.at[pl.ds(sid * 625 + i * ZROWS, ZROWS)])
    plsc.subcore_barrier()

    ibase = cid * N_EDGES + sid * EPT
    ebase = sid * EPT

    def group(g, carry):
        pltpu.sync_copy(aidx2.at[pl.ds(ibase + g * EPG, EPG)], big_a)
        pltpu.sync_copy(dst1.at[pl.ds(ebase + g * EPG, EPG)], big_d)

        def body(j, carry2):
            gbody = g * BPG + j

            @pl.when(gbody > 0)
            def _():
                for k in range(2):
                    S = sets[k]

                    @pl.when(is_c0)
                    def _(S=S):
                        pltpu.make_async_copy(S["b"], aggr_sh.at[S["id"]],
                                              S["s"]).wait()

                    @pl.when(~is_c0)
                    def _(S=S):
                        pltpu.make_async_copy(S["b"], g45o.at[pl.ds(0, CHUNK)],
                                              S["s"]).wait()

            gd = []
            idd = []
            for k in range(2):
                S = sets[k]
                off = (2 * j + k) * CHUNK
                idd.append(pltpu.async_copy(
                    dst1.at[pl.ds(ebase + g * EPG + off, CHUNK)],
                    S["id"], S["im"]))
                gd.append(pltpu.async_copy(
                    ta.at[big_a.at[pl.ds(off, CHUNK)]], S["a"], S["g"]))

                @pl.when(is_c0)
                def _(S=S, off=off):
                    pltpu.async_copy(
                        e2d.at[pl.ds(ebase + g * EPG + off, CHUNK)],
                        S["b"], S["g"])

                @pl.when(~is_c0)
                def _(S=S, off=off):
                    pltpu.async_copy(
                        t4tab.at[big_d.at[pl.ds(off, CHUNK)]], S["b"], S["g"])

            for k in range(2):
                S = sets[k]
                off = (2 * j + k) * CHUNK
                gd[k].wait()

                @pl.when(is_c0)
                def _(S=S):
                    pltpu.make_async_copy(e2d.at[pl.ds(0, CHUNK)], S["b"],
                                          S["g"]).wait()

                @pl.when(~is_c0)
                def _(S=S, off=off):
                    pltpu.make_async_copy(
                        t4tab.at[big_d.at[pl.ds(off, CHUNK)]], S["b"],
                        S["g"]).wait()

                av, bv = S["a"], S["b"]

                def _row(r, carry3):
                    for kk in range(H // 16):
                        sl = pl.ds(kk * 16, 16)
                        a = av[r, sl]
                        b = bv[r, sl]
                        bv[r, sl] = jnp.where(is_c0, a * b, a + b)
                    return carry3
                lax.fori_loop(0, CHUNK, _row, 0, unroll=2)
                idd[k].wait()

                @pl.when(is_c0)
                def _(S=S):
                    pltpu.async_copy(S["b"], aggr_sh.at[S["id"]], S["s"],
                                     add=True)

                @pl.when(~is_c0)
                def _(S=S, off=off):
                    pltpu.async_copy(
                        S["b"], g45o.at[pl.ds(ebase + g * EPG + off, CHUNK)],
                        S["s"])
            return carry2
        lax.fori_loop(0, BPG, body, 0)
        return carry
    lax.fori_loop(0, NGRP, group, 0)

    for k in range(2):
        S = sets[k]

        @pl.when(is_c0)
        def _(S=S):
            pltpu.make_async_copy(S["b"], aggr_sh.at[S["id"]], S["s"]).wait()

        @pl.when(~is_c0)
        def _(S=S):
            pltpu.make_async_copy(S["b"], g45o.at[pl.ds(0, CHUNK)],
                                  S["s"]).wait()

    plsc.subcore_barrier()

    @pl.when(is_c0 & (sid == 0))
    def _():
        pltpu.sync_copy(aggr_sh, aggro)


def _run_sc_edge(ta, t4tab, e2d, aidx2, dst1):
    mesh = plsc.VectorSubcoreMesh(core_axis_name="c", subcore_axis_name="s",
                                  num_cores=NC, num_subcores=NS)
    buf_set = [
        pltpu.VMEM((CHUNK,), jnp.int32),
        pltpu.VMEM((CHUNK, H), jnp.float32),
        pltpu.VMEM((CHUNK, H), jnp.float32),
    ]
    fn = pl.kernel(
        _sc_edge_body,
        out_type=[
            jax.ShapeDtypeStruct((N_EDGES, H), jnp.float32),
            jax.ShapeDtypeStruct((N_NODES, H), jnp.float32),
        ],
        mesh=mesh,
        scratch_types=(buf_set * 2) + [
            pltpu.VMEM((EPG,), jnp.int32),
            pltpu.VMEM((EPG,), jnp.int32),
            pltpu.VMEM((ZROWS, H), jnp.float32),
            pltpu.VMEM_SHARED((N_NODES, H), jnp.float32),
            pltpu.SemaphoreType.DMA,
            pltpu.SemaphoreType.DMA,
            pltpu.SemaphoreType.DMA,
            pltpu.SemaphoreType.DMA,
            pltpu.SemaphoreType.DMA,
            pltpu.SemaphoreType.DMA,
        ],
    )
    return fn(ta, t4tab, e2d, aidx2, dst1)


def _e_stats_body(e_ref, g45_ref, w3t_ref, sums_ref):
    i = pl.program_id(0)
    h = jnp.dot(e_ref[...], w3t_ref[...],
                preferred_element_type=jnp.float32) + g45_ref[...]

    @pl.when(i == 0)
    def _():
        sums_ref[...] = jnp.zeros_like(sums_ref)

    sums_ref[0, :] += jnp.sum(h, axis=0)
    sums_ref[1, :] += jnp.sum(h * h, axis=0)


def _e_stats(e, g45, w3t):
    grid = N_EDGES // EDGE_BLK
    return pl.pallas_call(
        _e_stats_body,
        grid=(grid,),
        in_specs=[
            pl.BlockSpec((EDGE_BLK, H), lambda i: (i, 0)),
            pl.BlockSpec((EDGE_BLK, H), lambda i: (i, 0)),
            pl.BlockSpec((H, H), lambda i: (0, 0)),
        ],
        out_specs=pl.BlockSpec((8, H), lambda i: (0, 0)),
        out_shape=jax.ShapeDtypeStruct((8, H), jnp.float32),
    )(e, g45, w3t)


def _e_apply_body(e_ref, g45_ref, w3t_ref, sums_ref, gamma_ref, beta_ref,
                  out_ref):
    h = jnp.dot(e_ref[...], w3t_ref[...],
                preferred_element_type=jnp.float32) + g45_ref[...]
    inv_n = 1.0 / N_EDGES
    mean = sums_ref[0:1, :] * inv_n
    var = sums_ref[1:2, :] * inv_n - mean * mean
    scale = gamma_ref[...] * lax.rsqrt(var + EPS)
    shift = beta_ref[...] - mean * scale
    out_ref[...] = e_ref[...] + jnp.maximum(h * scale + shift, 0.0)


def _e_apply(e, g45, w3t, sums, gamma, beta):
    grid = N_EDGES // EDGE_BLK
    return pl.pallas_call(
        _e_apply_body,
        grid=(grid,),
        in_specs=[
            pl.BlockSpec((EDGE_BLK, H), lambda i: (i, 0)),
            pl.BlockSpec((EDGE_BLK, H), lambda i: (i, 0)),
            pl.BlockSpec((H, H), lambda i: (0, 0)),
            pl.BlockSpec((8, H), lambda i: (0, 0)),
            pl.BlockSpec((1, H), lambda i: (0, 0)),
            pl.BlockSpec((1, H), lambda i: (0, 0)),
        ],
        out_specs=pl.BlockSpec((EDGE_BLK, H), lambda i: (i, 0)),
        out_shape=jax.ShapeDtypeStruct((N_EDGES, H), jnp.float32),
    )(e, g45, w3t, sums, gamma, beta)


def _x_stats_body(t1_ref, aggr_ref, sums_ref):
    i = pl.program_id(0)
    h = t1_ref[...] + aggr_ref[...]

    @pl.when(i == 0)
    def _():
        sums_ref[...] = jnp.zeros_like(sums_ref)

    sums_ref[0, :] += jnp.sum(h, axis=0)
    sums_ref[1, :] += jnp.sum(h * h, axis=0)


def _x_stats(t1, aggr):
    grid = N_NODES // NODE_BLK
    return pl.pallas_call(
        _x_stats_body,
        grid=(grid,),
        in_specs=[
            pl.BlockSpec((NODE_BLK, H), lambda i: (i, 0)),
            pl.BlockSpec((NODE_BLK, H), lambda i: (i, 0)),
        ],
        out_specs=pl.BlockSpec((8, H), lambda i: (0, 0)),
        out_shape=jax.ShapeDtypeStruct((8, H), jnp.float32),
    )(t1, aggr)


def _x_apply_body(x_ref, t1_ref, aggr_ref, sums_ref, gamma_ref, beta_ref,
                  out_ref):
    h = t1_ref[...] + aggr_ref[...]
    inv_n = 1.0 / N_NODES
    mean = sums_ref[0:1, :] * inv_n
    var = sums_ref[1:2, :] * inv_n - mean * mean
    scale = gamma_ref[...] * lax.rsqrt(var + EPS)
    shift = beta_ref[...] - mean * scale
    out_ref[...] = x_ref[...] + jnp.maximum(h * scale + shift, 0.0)


def _x_apply(x, t1, aggr, sums, gamma, beta):
    grid = N_NODES // NODE_BLK
    return pl.pallas_call(
        _x_apply_body,
        grid=(grid,),
        in_specs=[
            pl.BlockSpec((NODE_BLK, H), lambda i: (i, 0)),
            pl.BlockSpec((NODE_BLK, H), lambda i: (i, 0)),
            pl.BlockSpec((NODE_BLK, H), lambda i: (i, 0)),
            pl.BlockSpec((8, H), lambda i: (0, 0)),
            pl.BlockSpec((1, H), lambda i: (0, 0)),
            pl.BlockSpec((1, H), lambda i: (0, 0)),
        ],
        out_specs=pl.BlockSpec((NODE_BLK, H), lambda i: (i, 0)),
        out_shape=jax.ShapeDtypeStruct((N_NODES, H), jnp.float32),
    )(x, t1, aggr, sums, gamma, beta)


def kernel(x, e, W1, W2, W3, W4, W5, bn_x_gamma, bn_x_beta, bn_e_gamma,
           bn_e_beta, edge_index):
    src = edge_index[0].astype(jnp.int32)
    dst = edge_index[1].astype(jnp.int32)

    aidx2 = jnp.concatenate([src, src + N_NODES])

    ta3, t4, t1 = _make_tables(x, W2.T, W5.T, W4.T, W1.T)
    g45, aggr = _run_sc_edge(ta3.reshape(NC * N_NODES, H), t4, e, aidx2, dst)

    w3t = W3.T
    e_sums = _e_stats(e, g45, w3t)
    e_new = _e_apply(e, g45, w3t, e_sums,
                     bn_e_gamma.reshape(1, H), bn_e_beta.reshape(1, H))

    x_sums = _x_stats(t1, aggr)
    x_new = _x_apply(x, t1, aggr, x_sums,
                     bn_x_gamma.reshape(1, H), bn_x_beta.reshape(1, H))
    return (x_new, e_new)


# speedup vs baseline: 1.6065x; 1.0643x over previous
import jax
import jax.numpy as jnp
from jax import lax
from jax.experimental import pallas as pl
from jax.experimental.pallas import tpu as pltpu
from jax.experimental.pallas import tpu_sc as plsc

N_NODES = 10000
N_EDGES = 320000
H = 128
EPS = 1e-5

NC = 2
NS = 16
EPT = N_EDGES // NS
CHUNK = 80
CPT = EPT // CHUNK
EPG = 4000
BPG = EPG // (2 * CHUNK)
NGRP = EPT // EPG
ZROWS = 5

NODE_BLK = 1000
EDGE_BLK = 4000


def _tables_body(x_ref, w2t_ref, w5t_ref, w4t_ref, w1t_ref,
                 ta_ref, t4_ref, t1_ref):
    xb = x_ref[...]
    ta_ref[0] = jnp.dot(xb, w2t_ref[...], preferred_element_type=jnp.float32)
    ta_ref[1] = jnp.dot(xb, w5t_ref[...], preferred_element_type=jnp.float32)
    t4_ref[...] = jnp.dot(xb, w4t_ref[...], preferred_element_type=jnp.float32)
    t1_ref[...] = jnp.dot(xb, w1t_ref[...], preferred_element_type=jnp.float32)


def _make_tables(x, w2t, w5t, w4t, w1t):
    grid = N_NODES // NODE_BLK
    return pl.pallas_call(
        _tables_body,
        grid=(grid,),
        in_specs=[
            pl.BlockSpec((NODE_BLK, H), lambda i: (i, 0)),
            pl.BlockSpec((H, H), lambda i: (0, 0)),
            pl.BlockSpec((H, H), lambda i: (0, 0)),
            pl.BlockSpec((H, H), lambda i: (0, 0)),
            pl.BlockSpec((H, H), lambda i: (0, 0)),
        ],
        out_specs=[
            pl.BlockSpec((NC, NODE_BLK, H), lambda i: (0, i, 0)),
            pl.BlockSpec((NODE_BLK, H), lambda i: (i, 0)),
            pl.BlockSpec((NODE_BLK, H), lambda i: (i, 0)),
        ],
        out_shape=[
            jax.ShapeDtypeStruct((NC, N_NODES, H), jnp.float32),
            jax.ShapeDtypeStruct((N_NODES, H), jnp.float32),
            jax.ShapeDtypeStruct((N_NODES, H), jnp.float32),
        ],
    )(x, w2t, w5t, w4t, w1t)


def _sc_edge_body(ta, t4tab, e2d, aidx2, dst1, didx2, g45o, aggro,
                  id0, ab0, bb0,
                  id1, ab1, bb1,
                  big_a, big_d, zbuf, aggr_sh,
                  gsem0, gsem1, ssem0, ssem1, isem0, isem1):
    cid = lax.axis_index("c")
    sid = lax.axis_index("s")
    is_c0 = cid == 0

    sets = (
        dict(id=id0, a=ab0, b=bb0, g=gsem0, s=ssem0, im=isem0),
        dict(id=id1, a=ab1, b=bb1, g=gsem1, s=ssem1, im=isem1),
    )

    @pl.when(is_c0)
    def _():
        def _zrow(r, carry):
            for j in range(H // 16):
                zbuf[r, pl.ds(j * 16, 16)] = jnp.zeros((16,), jnp.float32)
            return carry
        lax.fori_loop(0, ZROWS, _zrow, 0)
        for i in range(625 // ZROWS):
            pltpu.sync_copy(zbuf, aggr_sh.at[pl.ds(sid * 625 + i * ZROWS, ZROWS)])
    plsc.subcore_barrier()

    ibase = cid * N_EDGES + sid * EPT
    ebase = sid * EPT

    def group(g, carry):
        pltpu.sync_copy(aidx2.at[pl.ds(ibase + g * EPG, EPG)], big_a)
        pltpu.sync_copy(dst1.at[pl.ds(ebase + g * EPG, EPG)], big_d)

        def body(j, carry2):
            gbody = g * BPG + j

            @pl.when(gbody > 0)
            def _():
                for k in range(2):
                    S = sets[k]

                    @pl.when(is_c0)
                    def _(S=S):
                        pltpu.make_async_copy(S["b"], aggr_sh.at[S["id"]],
                                              S["s"]).wait()

                    @pl.when(~is_c0)
                    def _(S=S):
                        pltpu.make_async_copy(S["b"], g45o.at[pl.ds(0, CHUNK)],
                                              S["s"]).wait()

            idd = []
            for k in range(2):
                S = sets[k]
                off = (2 * j + k) * CHUNK
                idd.append(pltpu.async_copy(
                    didx2.at[pl.ds(ibase + g * EPG + off, CHUNK)],
                    S["id"], S["im"]))

                @pl.when(is_c0)
                def _(S=S, off=off):
                    pltpu.async_copy(
                        ta.at[big_a.at[pl.ds(off, CHUNK)]], S["a"], S["g"])
                    pltpu.async_copy(
                        e2d.at[pl.ds(ebase + g * EPG + off, CHUNK)],
                        S["b"], S["g"])

                @pl.when(~is_c0)
                def _(S=S, off=off):
                    pltpu.async_copy(
                        t4tab.at[big_d.at[pl.ds(off, CHUNK)]], S["b"], S["g"])

            for k in range(2):
                S = sets[k]
                off = (2 * j + k) * CHUNK

                @pl.when(is_c0)
                def _(S=S, k=k, off=off):
                    pltpu.make_async_copy(
                        ta.at[big_a.at[pl.ds(off, CHUNK)]], S["a"],
                        S["g"]).wait()
                    pltpu.make_async_copy(e2d.at[pl.ds(0, CHUNK)], S["b"],
                                          S["g"]).wait()
                    av, bv = S["a"], S["b"]

                    def _row(r, carry3):
                        for kk in range(H // 16):
                            sl = pl.ds(kk * 16, 16)
                            bv[r, sl] = av[r, sl] * bv[r, sl]
                        return carry3
                    lax.fori_loop(0, CHUNK, _row, 0, unroll=4)
                    idd[k].wait()
                    pltpu.async_copy(S["b"], aggr_sh.at[S["id"]], S["s"],
                                     add=True)

                @pl.when(~is_c0)
                def _(S=S, k=k, off=off):
                    pltpu.make_async_copy(
                        t4tab.at[big_d.at[pl.ds(off, CHUNK)]], S["b"],
                        S["g"]).wait()
                    idd[k].wait()
                    pltpu.async_copy(ta.at[S["id"]], S["b"], S["g"], add=True)

            @pl.when(~is_c0)
            def _():
                for k in range(2):
                    S = sets[k]
                    off = (2 * j + k) * CHUNK
                    pltpu.make_async_copy(ta.at[S["id"]], S["b"],
                                          S["g"]).wait()
                    pltpu.async_copy(
                        S["b"], g45o.at[pl.ds(ebase + g * EPG + off, CHUNK)],
                        S["s"])
            return carry2
        lax.fori_loop(0, BPG, body, 0)
        return carry
    lax.fori_loop(0, NGRP, group, 0)

    for k in range(2):
        S = sets[k]

        @pl.when(is_c0)
        def _(S=S):
            pltpu.make_async_copy(S["b"], aggr_sh.at[S["id"]], S["s"]).wait()

        @pl.when(~is_c0)
        def _(S=S):
            pltpu.make_async_copy(S["b"], g45o.at[pl.ds(0, CHUNK)],
                                  S["s"]).wait()

    plsc.subcore_barrier()

    @pl.when(is_c0 & (sid == 0))
    def _():
        pltpu.sync_copy(aggr_sh, aggro)


def _run_sc_edge(ta, t4tab, e2d, aidx2, dst1, didx2):
    mesh = plsc.VectorSubcoreMesh(core_axis_name="c", subcore_axis_name="s",
                                  num_cores=NC, num_subcores=NS)
    buf_set = [
        pltpu.VMEM((CHUNK,), jnp.int32),
        pltpu.VMEM((CHUNK, H), jnp.float32),
        pltpu.VMEM((CHUNK, H), jnp.float32),
    ]
    fn = pl.kernel(
        _sc_edge_body,
        out_type=[
            jax.ShapeDtypeStruct((N_EDGES, H), jnp.float32),
            jax.ShapeDtypeStruct((N_NODES, H), jnp.float32),
        ],
        mesh=mesh,
        scratch_types=(buf_set * 2) + [
            pltpu.VMEM((EPG,), jnp.int32),
            pltpu.VMEM((EPG,), jnp.int32),
            pltpu.VMEM((ZROWS, H), jnp.float32),
            pltpu.VMEM_SHARED((N_NODES, H), jnp.float32),
            pltpu.SemaphoreType.DMA,
            pltpu.SemaphoreType.DMA,
            pltpu.SemaphoreType.DMA,
            pltpu.SemaphoreType.DMA,
            pltpu.SemaphoreType.DMA,
            pltpu.SemaphoreType.DMA,
        ],
    )
    return fn(ta, t4tab, e2d, aidx2, dst1, didx2)


def _e_stats_body(e_ref, g45_ref, w3t_ref, sums_ref):
    i = pl.program_id(0)
    h = jnp.dot(e_ref[...], w3t_ref[...],
                preferred_element_type=jnp.float32) + g45_ref[...]

    @pl.when(i == 0)
    def _():
        sums_ref[...] = jnp.zeros_like(sums_ref)

    sums_ref[0, :] += jnp.sum(h, axis=0)
    sums_ref[1, :] += jnp.sum(h * h, axis=0)


def _e_stats(e, g45, w3t):
    grid = N_EDGES // EDGE_BLK
    return pl.pallas_call(
        _e_stats_body,
        grid=(grid,),
        in_specs=[
            pl.BlockSpec((EDGE_BLK, H), lambda i: (i, 0)),
            pl.BlockSpec((EDGE_BLK, H), lambda i: (i, 0)),
            pl.BlockSpec((H, H), lambda i: (0, 0)),
        ],
        out_specs=pl.BlockSpec((8, H), lambda i: (0, 0)),
        out_shape=jax.ShapeDtypeStruct((8, H), jnp.float32),
    )(e, g45, w3t)


def _e_apply_body(e_ref, g45_ref, w3t_ref, sums_ref, gamma_ref, beta_ref,
                  out_ref):
    h = jnp.dot(e_ref[...], w3t_ref[...],
                preferred_element_type=jnp.float32) + g45_ref[...]
    inv_n = 1.0 / N_EDGES
    mean = sums_ref[0:1, :] * inv_n
    var = sums_ref[1:2, :] * inv_n - mean * mean
    scale = gamma_ref[...] * lax.rsqrt(var + EPS)
    shift = beta_ref[...] - mean * scale
    out_ref[...] = e_ref[...] + jnp.maximum(h * scale + shift, 0.0)


def _e_apply(e, g45, w3t, sums, gamma, beta):
    grid = N_EDGES // EDGE_BLK
    return pl.pallas_call(
        _e_apply_body,
        grid=(grid,),
        in_specs=[
            pl.BlockSpec((EDGE_BLK, H), lambda i: (i, 0)),
            pl.BlockSpec((EDGE_BLK, H), lambda i: (i, 0)),
            pl.BlockSpec((H, H), lambda i: (0, 0)),
            pl.BlockSpec((8, H), lambda i: (0, 0)),
            pl.BlockSpec((1, H), lambda i: (0, 0)),
            pl.BlockSpec((1, H), lambda i: (0, 0)),
        ],
        out_specs=pl.BlockSpec((EDGE_BLK, H), lambda i: (i, 0)),
        out_shape=jax.ShapeDtypeStruct((N_EDGES, H), jnp.float32),
    )(e, g45, w3t, sums, gamma, beta)


def _x_stats_body(t1_ref, aggr_ref, sums_ref):
    i = pl.program_id(0)
    h = t1_ref[...] + aggr_ref[...]

    @pl.when(i == 0)
    def _():
        sums_ref[...] = jnp.zeros_like(sums_ref)

    sums_ref[0, :] += jnp.sum(h, axis=0)
    sums_ref[1, :] += jnp.sum(h * h, axis=0)


def _x_stats(t1, aggr):
    grid = N_NODES // NODE_BLK
    return pl.pallas_call(
        _x_stats_body,
        grid=(grid,),
        in_specs=[
            pl.BlockSpec((NODE_BLK, H), lambda i: (i, 0)),
            pl.BlockSpec((NODE_BLK, H), lambda i: (i, 0)),
        ],
        out_specs=pl.BlockSpec((8, H), lambda i: (0, 0)),
        out_shape=jax.ShapeDtypeStruct((8, H), jnp.float32),
    )(t1, aggr)


def _x_apply_body(x_ref, t1_ref, aggr_ref, sums_ref, gamma_ref, beta_ref,
                  out_ref):
    h = t1_ref[...] + aggr_ref[...]
    inv_n = 1.0 / N_NODES
    mean = sums_ref[0:1, :] * inv_n
    var = sums_ref[1:2, :] * inv_n - mean * mean
    scale = gamma_ref[...] * lax.rsqrt(var + EPS)
    shift = beta_ref[...] - mean * scale
    out_ref[...] = x_ref[...] + jnp.maximum(h * scale + shift, 0.0)


def _x_apply(x, t1, aggr, sums, gamma, beta):
    grid = N_NODES // NODE_BLK
    return pl.pallas_call(
        _x_apply_body,
        grid=(grid,),
        in_specs=[
            pl.BlockSpec((NODE_BLK, H), lambda i: (i, 0)),
            pl.BlockSpec((NODE_BLK, H), lambda i: (i, 0)),
            pl.BlockSpec((NODE_BLK, H), lambda i: (i, 0)),
            pl.BlockSpec((8, H), lambda i: (0, 0)),
            pl.BlockSpec((1, H), lambda i: (0, 0)),
            pl.BlockSpec((1, H), lambda i: (0, 0)),
        ],
        out_specs=pl.BlockSpec((NODE_BLK, H), lambda i: (i, 0)),
        out_shape=jax.ShapeDtypeStruct((N_NODES, H), jnp.float32),
    )(x, t1, aggr, sums, gamma, beta)


def kernel(x, e, W1, W2, W3, W4, W5, bn_x_gamma, bn_x_beta, bn_e_gamma,
           bn_e_beta, edge_index):
    src = edge_index[0].astype(jnp.int32)
    dst = edge_index[1].astype(jnp.int32)

    aidx2 = jnp.concatenate([src, src + N_NODES])

    ta3, t4, t1 = _make_tables(x, W2.T, W5.T, W4.T, W1.T)
    didx2 = jnp.concatenate([dst, src + N_NODES])
    g45, aggr = _run_sc_edge(ta3.reshape(NC * N_NODES, H), t4, e, aidx2, dst,
                             didx2)

    w3t = W3.T
    e_sums = _e_stats(e, g45, w3t)
    e_new = _e_apply(e, g45, w3t, e_sums,
                     bn_e_gamma.reshape(1, H), bn_e_beta.reshape(1, H))

    x_sums = _x_stats(t1, aggr)
    x_new = _x_apply(x, t1, aggr, x_sums,
                     bn_x_gamma.reshape(1, H), bn_x_beta.reshape(1, H))
    return (x_new, e_new)


# speedup vs baseline: 1.6773x; 1.0441x over previous
import jax
import jax.numpy as jnp
from jax import lax
from jax.experimental import pallas as pl
from jax.experimental.pallas import tpu as pltpu
from jax.experimental.pallas import tpu_sc as plsc

N_NODES = 10000
N_EDGES = 320000
H = 128
EPS = 1e-5

NC = 2
NS = 16
EPT = N_EDGES // NS
CHUNK = 40
KSETS = 4
CPT = EPT // CHUNK
EPG = 4000
BPG = EPG // (KSETS * CHUNK)
NGRP = EPT // EPG
ZROWS = 5

NODE_BLK = 1000
EDGE_BLK = 4000


def _tables_body(x_ref, w2t_ref, w5t_ref, w4t_ref, w1t_ref,
                 ta_ref, t4_ref, t1_ref):
    xb = x_ref[...]
    ta_ref[0] = jnp.dot(xb, w2t_ref[...], preferred_element_type=jnp.float32)
    ta_ref[1] = jnp.dot(xb, w5t_ref[...], preferred_element_type=jnp.float32)
    t4_ref[...] = jnp.dot(xb, w4t_ref[...], preferred_element_type=jnp.float32)
    t1_ref[...] = jnp.dot(xb, w1t_ref[...], preferred_element_type=jnp.float32)


def _make_tables(x, w2t, w5t, w4t, w1t):
    grid = N_NODES // NODE_BLK
    return pl.pallas_call(
        _tables_body,
        grid=(grid,),
        in_specs=[
            pl.BlockSpec((NODE_BLK, H), lambda i: (i, 0)),
            pl.BlockSpec((H, H), lambda i: (0, 0)),
            pl.BlockSpec((H, H), lambda i: (0, 0)),
            pl.BlockSpec((H, H), lambda i: (0, 0)),
            pl.BlockSpec((H, H), lambda i: (0, 0)),
        ],
        out_specs=[
            pl.BlockSpec((NC, NODE_BLK, H), lambda i: (0, i, 0)),
            pl.BlockSpec((NODE_BLK, H), lambda i: (i, 0)),
            pl.BlockSpec((NODE_BLK, H), lambda i: (i, 0)),
        ],
        out_shape=[
            jax.ShapeDtypeStruct((NC, N_NODES, H), jnp.float32),
            jax.ShapeDtypeStruct((N_NODES, H), jnp.float32),
            jax.ShapeDtypeStruct((N_NODES, H), jnp.float32),
        ],
    )(x, w2t, w5t, w4t, w1t)


def _sc_edge_body(ta, t4tab, e2d, aidx2, dst1, didx2, g45o, aggro,
                  id0, ab0, bb0, id1, ab1, bb1,
                  id2, ab2, bb2, id3, ab3, bb3,
                  big_a, big_d, zbuf, aggr_sh,
                  gsem0, gsem1, gsem2, gsem3,
                  ssem0, ssem1, ssem2, ssem3,
                  isem0, isem1, isem2, isem3):
    cid = lax.axis_index("c")
    sid = lax.axis_index("s")
    is_c0 = cid == 0

    sets = (
        dict(id=id0, a=ab0, b=bb0, g=gsem0, s=ssem0, im=isem0),
        dict(id=id1, a=ab1, b=bb1, g=gsem1, s=ssem1, im=isem1),
        dict(id=id2, a=ab2, b=bb2, g=gsem2, s=ssem2, im=isem2),
        dict(id=id3, a=ab3, b=bb3, g=gsem3, s=ssem3, im=isem3),
    )

    @pl.when(is_c0)
    def _():
        def _zrow(r, carry):
            for j in range(H // 16):
                zbuf[r, pl.ds(j * 16, 16)] = jnp.zeros((16,), jnp.float32)
            return carry
        lax.fori_loop(0, ZROWS, _zrow, 0)
        for i in range(625 // ZROWS):
            pltpu.sync_copy(zbuf, aggr_sh.at[pl.ds(sid * 625 + i * ZROWS, ZROWS)])
    plsc.subcore_barrier()

    ibase = cid * N_EDGES + sid * EPT
    ebase = sid * EPT

    def group(g, carry):
        pltpu.sync_copy(aidx2.at[pl.ds(ibase + g * EPG, EPG)], big_a)
        pltpu.sync_copy(dst1.at[pl.ds(ebase + g * EPG, EPG)], big_d)

        def body(j, carry2):
            gbody = g * BPG + j

            @pl.when(gbody > 0)
            def _():
                for k in range(KSETS):
                    S = sets[k]

                    @pl.when(is_c0)
                    def _(S=S):
                        pltpu.make_async_copy(S["b"], aggr_sh.at[S["id"]],
                                              S["s"]).wait()

                    @pl.when(~is_c0)
                    def _(S=S):
                        pltpu.make_async_copy(S["b"], g45o.at[pl.ds(0, CHUNK)],
                                              S["s"]).wait()

            idd = []
            for k in range(KSETS):
                S = sets[k]
                off = (KSETS * j + k) * CHUNK
                idd.append(pltpu.async_copy(
                    didx2.at[pl.ds(ibase + g * EPG + off, CHUNK)],
                    S["id"], S["im"]))

                @pl.when(is_c0)
                def _(S=S, off=off):
                    pltpu.async_copy(
                        ta.at[big_a.at[pl.ds(off, CHUNK)]], S["a"], S["g"])
                    pltpu.async_copy(
                        e2d.at[pl.ds(ebase + g * EPG + off, CHUNK)],
                        S["b"], S["g"])

                @pl.when(~is_c0)
                def _(S=S, off=off):
                    pltpu.async_copy(
                        t4tab.at[big_d.at[pl.ds(off, CHUNK)]], S["b"], S["g"])

            for k in range(KSETS):
                S = sets[k]
                off = (KSETS * j + k) * CHUNK

                @pl.when(is_c0)
                def _(S=S, k=k, off=off):
                    pltpu.make_async_copy(
                        ta.at[big_a.at[pl.ds(off, CHUNK)]], S["a"],
                        S["g"]).wait()
                    pltpu.make_async_copy(e2d.at[pl.ds(0, CHUNK)], S["b"],
                                          S["g"]).wait()
                    av, bv = S["a"], S["b"]

                    def _row(r, carry3):
                        for kk in range(H // 16):
                            sl = pl.ds(kk * 16, 16)
                            bv[r, sl] = av[r, sl] * bv[r, sl]
                        return carry3
                    lax.fori_loop(0, CHUNK, _row, 0, unroll=4)
                    idd[k].wait()
                    pltpu.async_copy(S["b"], aggr_sh.at[S["id"]], S["s"],
                                     add=True)

                @pl.when(~is_c0)
                def _(S=S, k=k, off=off):
                    pltpu.make_async_copy(
                        t4tab.at[big_d.at[pl.ds(off, CHUNK)]], S["b"],
                        S["g"]).wait()
                    idd[k].wait()
                    pltpu.async_copy(ta.at[S["id"]], S["b"], S["g"], add=True)

            @pl.when(~is_c0)
            def _():
                for k in range(KSETS):
                    S = sets[k]
                    off = (KSETS * j + k) * CHUNK
                    pltpu.make_async_copy(ta.at[S["id"]], S["b"],
                                          S["g"]).wait()
                    pltpu.async_copy(
                        S["b"], g45o.at[pl.ds(ebase + g * EPG + off, CHUNK)],
                        S["s"])
            return carry2
        lax.fori_loop(0, BPG, body, 0)
        return carry
    lax.fori_loop(0, NGRP, group, 0)

    for k in range(KSETS):
        S = sets[k]

        @pl.when(is_c0)
        def _(S=S):
            pltpu.make_async_copy(S["b"], aggr_sh.at[S["id"]], S["s"]).wait()

        @pl.when(~is_c0)
        def _(S=S):
            pltpu.make_async_copy(S["b"], g45o.at[pl.ds(0, CHUNK)],
                                  S["s"]).wait()

    plsc.subcore_barrier()

    @pl.when(is_c0 & (sid == 0))
    def _():
        pltpu.sync_copy(aggr_sh, aggro)


def _run_sc_edge(ta, t4tab, e2d, aidx2, dst1, didx2):
    mesh = plsc.VectorSubcoreMesh(core_axis_name="c", subcore_axis_name="s",
                                  num_cores=NC, num_subcores=NS)
    buf_set = [
        pltpu.VMEM((CHUNK,), jnp.int32),
        pltpu.VMEM((CHUNK, H), jnp.float32),
        pltpu.VMEM((CHUNK, H), jnp.float32),
    ]
    fn = pl.kernel(
        _sc_edge_body,
        out_type=[
            jax.ShapeDtypeStruct((N_EDGES, H), jnp.float32),
            jax.ShapeDtypeStruct((N_NODES, H), jnp.float32),
        ],
        mesh=mesh,
        scratch_types=(buf_set * 4) + [
            pltpu.VMEM((EPG,), jnp.int32),
            pltpu.VMEM((EPG,), jnp.int32),
            pltpu.VMEM((ZROWS, H), jnp.float32),
            pltpu.VMEM_SHARED((N_NODES, H), jnp.float32),
        ] + [pltpu.SemaphoreType.DMA] * 12,
    )
    return fn(ta, t4tab, e2d, aidx2, dst1, didx2)


def _e_stats_body(e_ref, g45_ref, w3t_ref, sums_ref):
    i = pl.program_id(0)
    h = jnp.dot(e_ref[...], w3t_ref[...],
                preferred_element_type=jnp.float32) + g45_ref[...]

    @pl.when(i == 0)
    def _():
        sums_ref[...] = jnp.zeros_like(sums_ref)

    sums_ref[0, :] += jnp.sum(h, axis=0)
    sums_ref[1, :] += jnp.sum(h * h, axis=0)


def _e_stats(e, g45, w3t):
    grid = N_EDGES // EDGE_BLK
    return pl.pallas_call(
        _e_stats_body,
        grid=(grid,),
        in_specs=[
            pl.BlockSpec((EDGE_BLK, H), lambda i: (i, 0)),
            pl.BlockSpec((EDGE_BLK, H), lambda i: (i, 0)),
            pl.BlockSpec((H, H), lambda i: (0, 0)),
        ],
        out_specs=pl.BlockSpec((8, H), lambda i: (0, 0)),
        out_shape=jax.ShapeDtypeStruct((8, H), jnp.float32),
    )(e, g45, w3t)


def _e_apply_body(e_ref, g45_ref, w3t_ref, sums_ref, gamma_ref, beta_ref,
                  out_ref):
    h = jnp.dot(e_ref[...], w3t_ref[...],
                preferred_element_type=jnp.float32) + g45_ref[...]
    inv_n = 1.0 / N_EDGES
    mean = sums_ref[0:1, :] * inv_n
    var = sums_ref[1:2, :] * inv_n - mean * mean
    scale = gamma_ref[...] * lax.rsqrt(var + EPS)
    shift = beta_ref[...] - mean * scale
    out_ref[...] = e_ref[...] + jnp.maximum(h * scale + shift, 0.0)


def _e_apply(e, g45, w3t, sums, gamma, beta):
    grid = N_EDGES // EDGE_BLK
    return pl.pallas_call(
        _e_apply_body,
        grid=(grid,),
        in_specs=[
            pl.BlockSpec((EDGE_BLK, H), lambda i: (i, 0)),
            pl.BlockSpec((EDGE_BLK, H), lambda i: (i, 0)),
            pl.BlockSpec((H, H), lambda i: (0, 0)),
            pl.BlockSpec((8, H), lambda i: (0, 0)),
            pl.BlockSpec((1, H), lambda i: (0, 0)),
            pl.BlockSpec((1, H), lambda i: (0, 0)),
        ],
        out_specs=pl.BlockSpec((EDGE_BLK, H), lambda i: (i, 0)),
        out_shape=jax.ShapeDtypeStruct((N_EDGES, H), jnp.float32),
    )(e, g45, w3t, sums, gamma, beta)


def _x_stats_body(t1_ref, aggr_ref, sums_ref):
    i = pl.program_id(0)
    h = t1_ref[...] + aggr_ref[...]

    @pl.when(i == 0)
    def _():
        sums_ref[...] = jnp.zeros_like(sums_ref)

    sums_ref[0, :] += jnp.sum(h, axis=0)
    sums_ref[1, :] += jnp.sum(h * h, axis=0)


def _x_stats(t1, aggr):
    grid = N_NODES // NODE_BLK
    return pl.pallas_call(
        _x_stats_body,
        grid=(grid,),
        in_specs=[
            pl.BlockSpec((NODE_BLK, H), lambda i: (i, 0)),
            pl.BlockSpec((NODE_BLK, H), lambda i: (i, 0)),
        ],
        out_specs=pl.BlockSpec((8, H), lambda i: (0, 0)),
        out_shape=jax.ShapeDtypeStruct((8, H), jnp.float32),
    )(t1, aggr)


def _x_apply_body(x_ref, t1_ref, aggr_ref, sums_ref, gamma_ref, beta_ref,
                  out_ref):
    h = t1_ref[...] + aggr_ref[...]
    inv_n = 1.0 / N_NODES
    mean = sums_ref[0:1, :] * inv_n
    var = sums_ref[1:2, :] * inv_n - mean * mean
    scale = gamma_ref[...] * lax.rsqrt(var + EPS)
    shift = beta_ref[...] - mean * scale
    out_ref[...] = x_ref[...] + jnp.maximum(h * scale + shift, 0.0)


def _x_apply(x, t1, aggr, sums, gamma, beta):
    grid = N_NODES // NODE_BLK
    return pl.pallas_call(
        _x_apply_body,
        grid=(grid,),
        in_specs=[
            pl.BlockSpec((NODE_BLK, H), lambda i: (i, 0)),
            pl.BlockSpec((NODE_BLK, H), lambda i: (i, 0)),
            pl.BlockSpec((NODE_BLK, H), lambda i: (i, 0)),
            pl.BlockSpec((8, H), lambda i: (0, 0)),
            pl.BlockSpec((1, H), lambda i: (0, 0)),
            pl.BlockSpec((1, H), lambda i: (0, 0)),
        ],
        out_specs=pl.BlockSpec((NODE_BLK, H), lambda i: (i, 0)),
        out_shape=jax.ShapeDtypeStruct((N_NODES, H), jnp.float32),
    )(x, t1, aggr, sums, gamma, beta)


def kernel(x, e, W1, W2, W3, W4, W5, bn_x_gamma, bn_x_beta, bn_e_gamma,
           bn_e_beta, edge_index):
    src = edge_index[0].astype(jnp.int32)
    dst = edge_index[1].astype(jnp.int32)

    aidx2 = jnp.concatenate([src, src + N_NODES])

    ta3, t4, t1 = _make_tables(x, W2.T, W5.T, W4.T, W1.T)
    didx2 = jnp.concatenate([dst, src + N_NODES])
    g45, aggr = _run_sc_edge(ta3.reshape(NC * N_NODES, H), t4, e, aidx2, dst,
                             didx2)

    w3t = W3.T
    e_sums = _e_stats(e, g45, w3t)
    e_new = _e_apply(e, g45, w3t, e_sums,
                     bn_e_gamma.reshape(1, H), bn_e_beta.reshape(1, H))

    x_sums = _x_stats(t1, aggr)
    x_new = _x_apply(x, t1, aggr, x_sums,
                     bn_x_gamma.reshape(1, H), bn_x_beta.reshape(1, H))
    return (x_new, e_new)
